# baseline (speedup 1.0000x reference)
import functools

import jax
import jax.numpy as jnp
import numpy as np
from jax import lax
from jax.experimental import pallas as pl
from jax.experimental.pallas import tpu as pltpu
from jax.experimental.pallas import tpu_sc as plsc

_LVL_MAX = 20
_LVL = 5
_MAX = 2 ** _LVL_MAX
_BIAS = 2 ** (_LVL_MAX - _LVL) - 1
_NBITS = _LVL_MAX + 1

_BATCH = 16
_NPTS = 8192
_SUB = 64
_LANES = 128

_EPS_HI = np.float32(1e-6)
_EPS_LO = np.float32(1e-6 - float(np.float32(1e-6)))


def _two_sum(a, b):
    s = a + b
    bb = s - a
    err = (a - (s - bb)) + (b - bb)
    return s, err


def _fast_two_sum(a, b):
    s = a + b
    err = b - (s - a)
    return s, err


def _split(a):
    c = a * np.float32(4097.0)
    hi = c - (c - a)
    lo = a - hi
    return hi, lo


def _two_prod(a, b):
    p = a * b
    ah, al = _split(a)
    bh, bl = _split(b)
    err = ((ah * bh - p) + ah * bl + al * bh) + al * bl
    return p, err


def _ds_add(ah, al, bh, bl):
    s, e = _two_sum(ah, bh)
    e = e + (al + bl)
    return _fast_two_sum(s, e)


def _ds_mul(ah, al, bh, bl):
    p, e = _two_prod(ah, bh)
    e = e + (ah * bl + al * bh)
    return _fast_two_sum(p, e)


def _ds_div(ah, al, bh, bl):
    q0 = ah / bh
    ph, pe = _two_prod(q0, bh)
    pe = pe + q0 * bl
    rh, rl = _ds_add(ah, al, -ph, -pe)
    q1 = rh / bh
    return _fast_two_sum(q0, q1)


def _ds_sqrt(ah, al):
    s0 = jnp.sqrt(ah)
    ph, pe = _two_prod(s0, s0)
    rh, _ = _ds_add(ah, al, -ph, -pe)
    s1 = rh / (np.float32(2.0) * s0)
    return _fast_two_sum(s0, s1)


def _ds_fold_sum(h, l):
    n = h.shape[0]
    while n > 1:
        m = n // 2
        h, l = _ds_add(h[:m], l[:m], h[m:], l[m:])
        n = m
    n = h.shape[1]
    while n > 1:
        m = n // 2
        h, l = _ds_add(h[:, :m], l[:, :m], h[:, m:], l[:, m:])
        n = m
    return h, l


def _ds_fold_max(h, l):
    def pick(h1, l1, h2, l2):
        gt = (h1 > h2) | ((h1 == h2) & (l1 > l2))
        return jnp.where(gt, h1, h2), jnp.where(gt, l1, l2)

    n = h.shape[0]
    while n > 1:
        m = n // 2
        h, l = pick(h[:m], l[:m], h[m:], l[m:])
        n = m
    n = h.shape[1]
    while n > 1:
        m = n // 2
        h, l = pick(h[:, :m], l[:, :m], h[:, m:], l[:, m:])
        n = m
    return h, l


def _keys_math(x, y, w):
    coords = (x, y, w)

    means = []
    for cview in coords:
        sh, sl = _ds_fold_sum(cview, jnp.zeros_like(cview))
        inv = np.float32(1.0 / _NPTS)
        means.append((sh * inv, sl * inv))

    ds_d = []
    sqh = None
    for cview, (mh, ml) in zip(coords, means):
        dh, dl = _two_sum(cview, -mh)
        dl = dl - ml
        dh, dl = _fast_two_sum(dh, dl)
        ds_d.append((dh, dl))
        s2h, s2l = _ds_mul(dh, dl, dh, dl)
        if sqh is None:
            sqh, sql = s2h, s2l
        else:
            sqh, sql = _ds_add(sqh, sql, s2h, s2l)

    mh_, ml_ = _ds_fold_max(sqh, sql)
    rh, rl = _ds_sqrt(mh_, ml_)
    rh, rl = _ds_add(rh, rl, jnp.full_like(mh_, _EPS_HI), jnp.full_like(mh_, _EPS_LO))

    scale = np.float32(2.0 ** (_LVL_MAX - 1))
    cvals = []
    for (dh, dl) in ds_d:
        nh, nl = _ds_add(dh, dl, rh, rl)
        qh, ql = _ds_div(nh, nl, rh, rl)
        vh = qh * scale
        vl = ql * scale
        f = jnp.floor(vh)
        g = (vh - f) + vl
        ci = f.astype(jnp.int32) + jnp.floor(g).astype(jnp.int32)
        cvals.append(ci + np.int32(_BIAS))

    c0, c1, c2 = cvals

    for q in range(_NBITS - 1, 0, -1):
        Q = np.int32(1 << q)
        P = np.int32((1 << q) - 1)
        cond = (c0 & Q) != 0
        c0 = jnp.where(cond, c0 ^ P, c0)
        for which in (1, 2):
            ci = c1 if which == 1 else c2
            cond = (ci & Q) != 0
            t = (c0 ^ ci) & P
            new_c0 = jnp.where(cond, c0 ^ P, c0 ^ t)
            new_ci = jnp.where(cond, ci, ci ^ t)
            c0 = new_c0
            if which == 1:
                c1 = new_ci
            else:
                c2 = new_ci

    c1 = c1 ^ c0
    c2 = c2 ^ c1
    t = jnp.zeros_like(c0)
    for q in range(_NBITS - 1, 0, -1):
        Q = np.int32(1 << q)
        P = np.int32((1 << q) - 1)
        t = jnp.where((c2 & Q) != 0, t ^ P, t)
    c0 = c0 ^ t
    c1 = c1 ^ t
    c2 = c2 ^ t

    hi = jnp.zeros_like(c0)
    lo = jnp.zeros_like(c0)
    cs = (c0, c1, c2)
    for q in range(_NBITS):
        for i in range(3):
            pos = 3 * q + 2 - i
            bit = (cs[i] >> np.int32(q)) & np.int32(1)
            if pos >= 32:
                hi = hi | (bit << np.int32(pos - 32))
            else:
                lo = lo | (bit << np.int32(pos))

    return hi, lo


_TCB = 2


def _key_body(z_ref, hi_ref, lo_ref):
    for k in range(_TCB):
        hi, lo = _keys_math(z_ref[0, k], z_ref[1, k], z_ref[2, k])
        hi_ref[k] = hi
        lo_ref[k] = lo


def _hilbert_keys(zt):
    return pl.pallas_call(
        _key_body,
        grid=(_BATCH // _TCB,),
        in_specs=[
            pl.BlockSpec((3, _TCB, _SUB, _LANES), lambda b: (0, b, 0, 0)),
        ],
        out_specs=[
            pl.BlockSpec((_TCB, _SUB, _LANES), lambda b: (b, 0, 0)),
            pl.BlockSpec((_TCB, _SUB, _LANES), lambda b: (b, 0, 0)),
        ],
        out_shape=[
            jax.ShapeDtypeStruct((_BATCH, _SUB, _LANES), jnp.int32),
            jax.ShapeDtypeStruct((_BATCH, _SUB, _LANES), jnp.int32),
        ],
    )(zt)


_NBLK = 4
_BLK = _NPTS // _NBLK
_NVEC = _BLK // 16
_DIGW = 13
_RADIX = 1 << _DIGW
_NPASS = -(-63 // _DIGW)


def _sort_body(hi_hbm, lo_hbm, pa_hbm, re_hbm,
               ahi, alo, aval, bhi, blo, bval, occ, *hists):
    c = lax.axis_index("c")
    s = lax.axis_index("s")
    wid = s * np.int32(2) + c

    @pl.when(wid < _BATCH)
    def _():
        b = wid
        pltpu.sync_copy(hi_hbm.at[b], ahi)
        pltpu.sync_copy(lo_hbm.at[b], alo)

        lane = lax.iota(jnp.int32, 16)
        zeros = jnp.zeros((16,), jnp.int32)

        bufs = ((ahi, alo, aval), (bhi, blo, bval))
        last_lo = max(p for p in range(_NPASS) if _DIGW * p < 32)
        for p in range(_NPASS):
            shi, slo, sval = bufs[p % 2]
            dhi, dlo, dval = bufs[1 - p % 2]
            lobit = _DIGW * p
            dmask = np.int32(_RADIX - 1)
            need_lo = lobit < 32
            need_hi = lobit + _DIGW > 32
            carry_lo = p < last_lo
            carry_hi = p < _NPASS - 1
            last = p == _NPASS - 1

            def get_digit(off):
                wlo = slo[pl.ds(off, 16)] if need_lo else None
                whi = shi[pl.ds(off, 16)] if need_hi else None
                if not need_hi:
                    d = (wlo >> np.int32(lobit)) & dmask
                elif not need_lo:
                    d = (whi >> np.int32(lobit - 32)) & dmask
                else:
                    nlo = 32 - lobit
                    d = ((wlo >> np.int32(lobit)) & np.int32((1 << nlo) - 1)) \
                        | ((whi & np.int32((1 << (_DIGW - nlo)) - 1))
                           << np.int32(nlo))
                return d, wlo, whi

            def zbody(j, _):
                off = j * np.int32(16)
                for u in range(_NBLK):
                    hists[u][pl.ds(off, 16)] = zeros
                return np.int32(0)
            lax.fori_loop(np.int32(0), np.int32(_RADIX // 16), zbody,
                          np.int32(0))

            def hbody(j, _):
                for u in range(_NBLK):
                    off = j * np.int32(16) + np.int32(u * _BLK)
                    d, _wl, _wh = get_digit(off)
                    occv, lastm = plsc.scan_count(d)
                    cnt = plsc.load_gather(hists[u], [d])
                    plsc.store_scatter(hists[u], [d], cnt + occv, mask=lastm)
                    occ[pl.ds(off, 16)] = cnt + occv - np.int32(1)
                return np.int32(0)
            lax.fori_loop(np.int32(0), np.int32(_NVEC), hbody, np.int32(0))

            def sbody(j, carry):
                off = j * np.int32(16)
                es = [h[pl.ds(off, 16)] for h in hists]
                pref = [es[0]]
                for u in range(1, _NBLK):
                    pref.append(pref[-1] + es[u])
                t = pref[-1]
                incl = plsc.cumsum(t)
                base = incl - t + carry
                hists[0][pl.ds(off, 16)] = base
                for u in range(1, _NBLK):
                    hists[u][pl.ds(off, 16)] = base + pref[u - 1]
                return carry + jnp.sum(t, dtype=jnp.int32)
            lax.fori_loop(np.int32(0), np.int32(_RADIX // 16), sbody,
                          jnp.int32(0))

            def pbody(j, _):
                for u in range(_NBLK):
                    off = j * np.int32(16) + np.int32(u * _BLK)
                    d, wlo, whi = get_digit(off)
                    base = plsc.load_gather(hists[u], [d])
                    rank = base + occ[pl.ds(off, 16)]
                    if p == 0:
                        kv = lane + off
                    else:
                        kv = sval[pl.ds(off, 16)]
                    if carry_hi:
                        v = whi if whi is not None else shi[pl.ds(off, 16)]
                        plsc.store_scatter(dhi, [rank], v)
                    if carry_lo:
                        v = wlo if wlo is not None else slo[pl.ds(off, 16)]
                        plsc.store_scatter(dlo, [rank], v)
                    plsc.store_scatter(dval, [rank], kv)
                    if last:
                        plsc.store_scatter(dhi, [kv], rank)
                return np.int32(0)
            lax.fori_loop(np.int32(0), np.int32(_NVEC), pbody, np.int32(0))

        fin_hi, _fin_lo, fin_val = bufs[_NPASS % 2]
        pltpu.sync_copy(fin_val, pa_hbm.at[b])
        pltpu.sync_copy(fin_hi, re_hbm.at[b])


@functools.lru_cache(maxsize=1)
def _sort_call():
    return pl.kernel(
        _sort_body,
        out_type=[
            jax.ShapeDtypeStruct((_BATCH, _NPTS), jnp.int32),
            jax.ShapeDtypeStruct((_BATCH, _NPTS), jnp.int32),
        ],
        mesh=plsc.VectorSubcoreMesh(core_axis_name="c", subcore_axis_name="s"),
        scratch_types=[
            pltpu.VMEM((_NPTS,), jnp.int32),
            pltpu.VMEM((_NPTS,), jnp.int32),
            pltpu.VMEM((_NPTS,), jnp.int32),
            pltpu.VMEM((_NPTS,), jnp.int32),
            pltpu.VMEM((_NPTS,), jnp.int32),
            pltpu.VMEM((_NPTS,), jnp.int32),
            pltpu.VMEM((_NPTS,), jnp.int32),
        ] + [pltpu.VMEM((_RADIX,), jnp.int32) for _ in range(_NBLK)],
        compiler_params=pltpu.CompilerParams(needs_layout_passes=False),
    )


def kernel(z):
    with jax.enable_x64(False):
        zt = jnp.transpose(z, (2, 0, 1)).reshape(3, _BATCH, _SUB, _LANES)
        hi, lo = _hilbert_keys(zt)
        hi = hi.reshape(_BATCH, _NPTS)
        lo = lo.reshape(_BATCH, _NPTS)
        pa, re = _sort_call()(hi, lo)
    return pa.astype(jnp.int64), re.astype(jnp.int64)


# speedup vs baseline: 1.1171x; 1.1171x over previous
import functools

import jax
import jax.numpy as jnp
import numpy as np
from jax import lax
from jax.experimental import pallas as pl
from jax.experimental.pallas import tpu as pltpu
from jax.experimental.pallas import tpu_sc as plsc

_LVL_MAX = 20
_LVL = 5
_MAX = 2 ** _LVL_MAX
_BIAS = 2 ** (_LVL_MAX - _LVL) - 1
_NBITS = _LVL_MAX + 1

_BATCH = 16
_NPTS = 8192
_SUB = 64
_LANES = 128

_EPS_HI = np.float32(1e-6)
_EPS_LO = np.float32(1e-6 - float(np.float32(1e-6)))


def _two_sum(a, b):
    s = a + b
    bb = s - a
    err = (a - (s - bb)) + (b - bb)
    return s, err


def _fast_two_sum(a, b):
    s = a + b
    err = b - (s - a)
    return s, err


def _split(a):
    c = a * np.float32(4097.0)
    hi = c - (c - a)
    lo = a - hi
    return hi, lo


def _two_prod(a, b):
    p = a * b
    ah, al = _split(a)
    bh, bl = _split(b)
    err = ((ah * bh - p) + ah * bl + al * bh) + al * bl
    return p, err


def _ds_add(ah, al, bh, bl):
    s, e = _two_sum(ah, bh)
    e = e + (al + bl)
    return _fast_two_sum(s, e)


def _ds_mul(ah, al, bh, bl):
    p, e = _two_prod(ah, bh)
    e = e + (ah * bl + al * bh)
    return _fast_two_sum(p, e)


def _ds_div(ah, al, bh, bl):
    q0 = ah / bh
    ph, pe = _two_prod(q0, bh)
    pe = pe + q0 * bl
    rh, rl = _ds_add(ah, al, -ph, -pe)
    q1 = rh / bh
    return _fast_two_sum(q0, q1)


def _ds_sqrt(ah, al):
    s0 = jnp.sqrt(ah)
    ph, pe = _two_prod(s0, s0)
    rh, _ = _ds_add(ah, al, -ph, -pe)
    s1 = rh / (np.float32(2.0) * s0)
    return _fast_two_sum(s0, s1)


def _ds_fold_sum(h, l):
    n = h.shape[0]
    while n > 1:
        m = n // 2
        h, l = _ds_add(h[:m], l[:m], h[m:], l[m:])
        n = m
    n = h.shape[1]
    while n > 1:
        m = n // 2
        h, l = _ds_add(h[:, :m], l[:, :m], h[:, m:], l[:, m:])
        n = m
    return h, l


def _ds_fold_max(h, l):
    def pick(h1, l1, h2, l2):
        gt = (h1 > h2) | ((h1 == h2) & (l1 > l2))
        return jnp.where(gt, h1, h2), jnp.where(gt, l1, l2)

    n = h.shape[0]
    while n > 1:
        m = n // 2
        h, l = pick(h[:m], l[:m], h[m:], l[m:])
        n = m
    n = h.shape[1]
    while n > 1:
        m = n // 2
        h, l = pick(h[:, :m], l[:, :m], h[:, m:], l[:, m:])
        n = m
    return h, l


def _keys_math(x, y, w):
    coords = (x, y, w)

    means = []
    for cview in coords:
        sh, sl = _ds_fold_sum(cview, jnp.zeros_like(cview))
        inv = np.float32(1.0 / _NPTS)
        means.append((sh * inv, sl * inv))

    ds_d = []
    sqh = None
    for cview, (mh, ml) in zip(coords, means):
        dh, dl = _two_sum(cview, -mh)
        dl = dl - ml
        dh, dl = _fast_two_sum(dh, dl)
        ds_d.append((dh, dl))
        s2h, s2l = _ds_mul(dh, dl, dh, dl)
        if sqh is None:
            sqh, sql = s2h, s2l
        else:
            sqh, sql = _ds_add(sqh, sql, s2h, s2l)

    mh_, ml_ = _ds_fold_max(sqh, sql)
    rh, rl = _ds_sqrt(mh_, ml_)
    rh, rl = _ds_add(rh, rl, jnp.full_like(mh_, _EPS_HI), jnp.full_like(mh_, _EPS_LO))

    scale = np.float32(2.0 ** (_LVL_MAX - 1))
    cvals = []
    for (dh, dl) in ds_d:
        nh, nl = _ds_add(dh, dl, rh, rl)
        qh, ql = _ds_div(nh, nl, rh, rl)
        vh = qh * scale
        vl = ql * scale
        f = jnp.floor(vh)
        g = (vh - f) + vl
        ci = f.astype(jnp.int32) + jnp.floor(g).astype(jnp.int32)
        cvals.append(ci + np.int32(_BIAS))

    c0, c1, c2 = cvals

    for q in range(_NBITS - 1, 0, -1):
        Q = np.int32(1 << q)
        P = np.int32((1 << q) - 1)
        cond = (c0 & Q) != 0
        c0 = jnp.where(cond, c0 ^ P, c0)
        for which in (1, 2):
            ci = c1 if which == 1 else c2
            cond = (ci & Q) != 0
            t = (c0 ^ ci) & P
            new_c0 = jnp.where(cond, c0 ^ P, c0 ^ t)
            new_ci = jnp.where(cond, ci, ci ^ t)
            c0 = new_c0
            if which == 1:
                c1 = new_ci
            else:
                c2 = new_ci

    c1 = c1 ^ c0
    c2 = c2 ^ c1
    t = jnp.zeros_like(c0)
    for q in range(_NBITS - 1, 0, -1):
        Q = np.int32(1 << q)
        P = np.int32((1 << q) - 1)
        t = jnp.where((c2 & Q) != 0, t ^ P, t)
    c0 = c0 ^ t
    c1 = c1 ^ t
    c2 = c2 ^ t

    hi = jnp.zeros_like(c0)
    lo = jnp.zeros_like(c0)
    cs = (c0, c1, c2)
    for q in range(_NBITS):
        for i in range(3):
            pos = 3 * q + 2 - i
            bit = (cs[i] >> np.int32(q)) & np.int32(1)
            if pos >= 32:
                hi = hi | (bit << np.int32(pos - 32))
            else:
                lo = lo | (bit << np.int32(pos))

    return hi, lo


_TCB = 4


def _key_body(z_ref, hi_ref, lo_ref):
    for k in range(_TCB):
        hi, lo = _keys_math(z_ref[0, k], z_ref[1, k], z_ref[2, k])
        hi_ref[k] = hi
        lo_ref[k] = lo


def _hilbert_keys(zt):
    return pl.pallas_call(
        _key_body,
        grid=(_BATCH // _TCB,),
        in_specs=[
            pl.BlockSpec((3, _TCB, _SUB, _LANES), lambda b: (0, b, 0, 0)),
        ],
        out_specs=[
            pl.BlockSpec((_TCB, _SUB, _LANES), lambda b: (b, 0, 0)),
            pl.BlockSpec((_TCB, _SUB, _LANES), lambda b: (b, 0, 0)),
        ],
        out_shape=[
            jax.ShapeDtypeStruct((_BATCH, _SUB, _LANES), jnp.int32),
            jax.ShapeDtypeStruct((_BATCH, _SUB, _LANES), jnp.int32),
        ],
    )(zt)


_NBLK = 4
_BLK = _NPTS // _NBLK
_NVEC = _BLK // 16
_DIGW = 11
_RADIX = 1 << _DIGW
_NPASS = -(-63 // _DIGW)


def _sort_body(hi_hbm, lo_hbm, pa_hbm, re_hbm,
               ahi, alo, aval, bhi, blo, bval, occ, *hists):
    c = lax.axis_index("c")
    s = lax.axis_index("s")
    wid = s * np.int32(2) + c

    @pl.when(wid < _BATCH)
    def _():
        b = wid
        pltpu.sync_copy(hi_hbm.at[b], ahi)
        pltpu.sync_copy(lo_hbm.at[b], alo)

        lane = lax.iota(jnp.int32, 16)
        zeros = jnp.zeros((16,), jnp.int32)

        bufs = ((ahi, alo, aval), (bhi, blo, bval))
        last_lo = max(p for p in range(_NPASS) if _DIGW * p < 32)
        for p in range(_NPASS):
            shi, slo, sval = bufs[p % 2]
            dhi, dlo, dval = bufs[1 - p % 2]
            lobit = _DIGW * p
            dmask = np.int32(_RADIX - 1)
            need_lo = lobit < 32
            need_hi = lobit + _DIGW > 32
            carry_lo = p < last_lo
            carry_hi = p < _NPASS - 1
            last = p == _NPASS - 1

            def get_digit(off):
                wlo = slo[pl.ds(off, 16)] if need_lo else None
                whi = shi[pl.ds(off, 16)] if need_hi else None
                if not need_hi:
                    d = (wlo >> np.int32(lobit)) & dmask
                elif not need_lo:
                    d = (whi >> np.int32(lobit - 32)) & dmask
                else:
                    nlo = 32 - lobit
                    d = ((wlo >> np.int32(lobit)) & np.int32((1 << nlo) - 1)) \
                        | ((whi & np.int32((1 << (_DIGW - nlo)) - 1))
                           << np.int32(nlo))
                return d, wlo, whi

            def zbody(j, _):
                off = j * np.int32(16)
                for u in range(_NBLK):
                    hists[u][pl.ds(off, 16)] = zeros
                return np.int32(0)
            lax.fori_loop(np.int32(0), np.int32(_RADIX // 16), zbody,
                          np.int32(0))

            def hbody(j, _):
                for u in range(_NBLK):
                    off = j * np.int32(16) + np.int32(u * _BLK)
                    d, _wl, _wh = get_digit(off)
                    occv, lastm = plsc.scan_count(d)
                    cnt = plsc.load_gather(hists[u], [d])
                    plsc.store_scatter(hists[u], [d], cnt + occv, mask=lastm)
                    occ[pl.ds(off, 16)] = cnt + occv - np.int32(1)
                return np.int32(0)
            lax.fori_loop(np.int32(0), np.int32(_NVEC), hbody, np.int32(0))

            def sbody(j, carry):
                off = j * np.int32(16)
                es = [h[pl.ds(off, 16)] for h in hists]
                pref = [es[0]]
                for u in range(1, _NBLK):
                    pref.append(pref[-1] + es[u])
                t = pref[-1]
                incl = plsc.cumsum(t)
                base = incl - t + carry
                hists[0][pl.ds(off, 16)] = base
                for u in range(1, _NBLK):
                    hists[u][pl.ds(off, 16)] = base + pref[u - 1]
                return carry + jnp.sum(t, dtype=jnp.int32)
            lax.fori_loop(np.int32(0), np.int32(_RADIX // 16), sbody,
                          jnp.int32(0))

            def pbody(j, _):
                for u in range(_NBLK):
                    off = j * np.int32(16) + np.int32(u * _BLK)
                    d, wlo, whi = get_digit(off)
                    base = plsc.load_gather(hists[u], [d])
                    rank = base + occ[pl.ds(off, 16)]
                    if p == 0:
                        kv = lane + off
                    else:
                        kv = sval[pl.ds(off, 16)]
                    if carry_hi:
                        v = whi if whi is not None else shi[pl.ds(off, 16)]
                        plsc.store_scatter(dhi, [rank], v)
                    if carry_lo:
                        v = wlo if wlo is not None else slo[pl.ds(off, 16)]
                        plsc.store_scatter(dlo, [rank], v)
                    plsc.store_scatter(dval, [rank], kv)
                    if last:
                        plsc.store_scatter(dhi, [kv], rank)
                return np.int32(0)
            lax.fori_loop(np.int32(0), np.int32(_NVEC), pbody, np.int32(0))

        fin_hi, _fin_lo, fin_val = bufs[_NPASS % 2]
        pltpu.sync_copy(fin_val, pa_hbm.at[b])
        pltpu.sync_copy(fin_hi, re_hbm.at[b])


@functools.lru_cache(maxsize=1)
def _sort_call():
    return pl.kernel(
        _sort_body,
        out_type=[
            jax.ShapeDtypeStruct((_BATCH, _NPTS), jnp.int32),
            jax.ShapeDtypeStruct((_BATCH, _NPTS), jnp.int32),
        ],
        mesh=plsc.VectorSubcoreMesh(core_axis_name="c", subcore_axis_name="s"),
        scratch_types=[
            pltpu.VMEM((_NPTS,), jnp.int32),
            pltpu.VMEM((_NPTS,), jnp.int32),
            pltpu.VMEM((_NPTS,), jnp.int32),
            pltpu.VMEM((_NPTS,), jnp.int32),
            pltpu.VMEM((_NPTS,), jnp.int32),
            pltpu.VMEM((_NPTS,), jnp.int32),
            pltpu.VMEM((_NPTS,), jnp.int32),
        ] + [pltpu.VMEM((_RADIX,), jnp.int32) for _ in range(_NBLK)],
        compiler_params=pltpu.CompilerParams(needs_layout_passes=False),
    )


def kernel(z):
    with jax.enable_x64(False):
        zt = jnp.transpose(z, (2, 0, 1)).reshape(3, _BATCH, _SUB, _LANES)
        hi, lo = _hilbert_keys(zt)
        hi = hi.reshape(_BATCH, _NPTS)
        lo = lo.reshape(_BATCH, _NPTS)
        pa, re = _sort_call()(hi, lo)
    return pa.astype(jnp.int64), re.astype(jnp.int64)
